# baseline (speedup 1.0000x reference)
import functools

import jax
import jax.numpy as jnp
from jax import lax
from jax.experimental import pallas as pl
from jax.experimental.pallas import tpu as pltpu
from jax.experimental.pallas import tpu_sc as plsc

E = 8
K = 2
T = 2048
D = 1024
F = 1024
ROWS = T * K
BLK = 256
NB = 23
NPAD = NB * BLK

RR = 32
RL = 128

NC = 2
NS = 16
NW = NC * NS
TPW = T // NW


HD = D // 2


def _bf16_hi_bits(f):
    b = lax.bitcast_convert_type(f, jnp.uint32)
    return b + jnp.uint32(0x7FFF) + ((b >> 16) & jnp.uint32(1))


def _pack_cols(lo_f, hi_f):
    lo = _bf16_hi_bits(lo_f) >> 16
    hi = _bf16_hi_bits(hi_f) & jnp.uint32(0xFFFF0000)
    return lax.bitcast_convert_type(lo | hi, jnp.int32)


def _unpack_cols(p_i32):
    p = lax.bitcast_convert_type(p_i32, jnp.uint32)
    lo = lax.bitcast_convert_type(p << 16, jnp.float32)
    hi = lax.bitcast_convert_type(p & jnp.uint32(0xFFFF0000), jnp.float32)
    return lo, hi


def _routing_body(fe_ref, x_ref, counts_ref, dest_ref, be_ref, xi_ref):
    x = x_ref[...]
    xi_ref[:T, :] = _pack_cols(x[:, :HD], x[:, HD:])
    fe = fe_ref[...]
    dest = jnp.zeros((RR, RL), jnp.int32)
    counts = jnp.zeros((1, E), jnp.int32)
    bexp = jnp.zeros((1, NB), jnp.int32)
    lane_e = lax.broadcasted_iota(jnp.int32, (1, E), 1)
    lane_b = lax.broadcasted_iota(jnp.int32, (1, NB), 1)
    blk_start = jnp.int32(0)
    for e in range(E):
        m = (fe == e).astype(jnp.int32)
        ic = m
        for s in (1, 2, 4, 8, 16, 32, 64):
            ic = ic + jnp.concatenate(
                [jnp.zeros((RR, s), jnp.int32), ic[:, : RL - s]], axis=1
            )
        rt = ic[:, RL - 1 :]
        er = rt
        for s in (1, 2, 4, 8, 16):
            er = er + jnp.concatenate(
                [jnp.zeros((s, 1), jnp.int32), er[: RR - s, :]], axis=0
            )
        er = er - rt
        c_e = er[RR - 1, 0] + rt[RR - 1, 0]
        nblk_e = (c_e + BLK - 1) // BLK
        pad_base = blk_start * BLK
        rank_e = er + ic - 1
        dest = dest + m * (rank_e + pad_base)
        counts = counts + jnp.where(lane_e == e, c_e, 0)
        bexp = bexp + (lane_b >= blk_start).astype(jnp.int32)
        blk_start = blk_start + nblk_e
    counts_ref[...] = counts
    dest_ref[...] = dest
    be_ref[...] = jnp.clip(bexp - 1, 0, E - 1)


_routing_call = pl.pallas_call(
    _routing_body,
    out_shape=[
        jax.ShapeDtypeStruct((1, E), jnp.int32),
        jax.ShapeDtypeStruct((RR, RL), jnp.int32),
        jax.ShapeDtypeStruct((1, NB), jnp.int32),
        jax.ShapeDtypeStruct((NPAD, HD), jnp.int32),
    ],
)


def _routing(expert_indices, x):
    fe = expert_indices.astype(jnp.int32).T.reshape(RR, RL)
    counts, dest, block_expert, xi = _routing_call(fe, x)
    return counts.reshape(E), dest.reshape(ROWS), block_expert.reshape(NB), xi


RPW = ROWS // NW
CH = 64


@functools.lru_cache(maxsize=None)
def _sc_kernels():
    mesh = plsc.VectorSubcoreMesh(core_axis_name="c", subcore_axis_name="s")

    @functools.partial(
        pl.kernel,
        mesh=mesh,
        out_type=jax.ShapeDtypeStruct((NPAD, HD), jnp.int32),
        scratch_types=[
            pltpu.VMEM((CH,), jnp.int32),
            pltpu.VMEM((CH,), jnp.int32),
            pltpu.VMEM((CH, HD), jnp.int32),
            pltpu.SemaphoreType.DMA,
        ],
    )
    def permute(in_hbm, idxa_hbm, idxb_hbm, out_hbm, a_v, b_v, buf, sem):
        wid = lax.axis_index("s") * NC + lax.axis_index("c")
        base = wid * RPW
        for c in range(RPW // CH):
            off = base + c * CH
            pltpu.sync_copy(idxa_hbm.at[pl.ds(off, CH)], a_v)
            pltpu.sync_copy(idxb_hbm.at[pl.ds(off, CH)], b_v)
            pltpu.async_copy(in_hbm.at[a_v], buf, sem).wait()
            pltpu.async_copy(buf, out_hbm.at[b_v], sem).wait()

    return permute


def _gemm_body(be_ref, xs_ref, w1_ref, w2_ref, ys_ref):
    xl, xh = _unpack_cols(xs_ref[...])
    h = jnp.dot(xl, w1_ref[0, :HD, :], preferred_element_type=jnp.float32)
    h = h + jnp.dot(xh, w1_ref[0, HD:, :], preferred_element_type=jnp.float32)
    h = jnp.maximum(h, 0.0)
    y = jnp.dot(h, w2_ref[0], preferred_element_type=jnp.float32)
    ys_ref[...] = _pack_cols(y[:, :HD], y[:, HD:])


_grouped_gemm = pl.pallas_call(
    _gemm_body,
    grid_spec=pltpu.PrefetchScalarGridSpec(
        num_scalar_prefetch=1,
        grid=(NB,),
        in_specs=[
            pl.BlockSpec((BLK, HD), lambda b, be: (b, 0)),
            pl.BlockSpec((1, D, F), lambda b, be: (be[b], 0, 0)),
            pl.BlockSpec((1, F, D), lambda b, be: (be[b], 0, 0)),
        ],
        out_specs=pl.BlockSpec((BLK, HD), lambda b, be: (b, 0)),
    ),
    out_shape=jax.ShapeDtypeStruct((NPAD, HD), jnp.int32),
    compiler_params=pltpu.CompilerParams(
        dimension_semantics=("arbitrary",),
    ),
)


TBC = 256


def _combine_body(y0_ref, y1_ref, w_ref, out_ref):
    w = w_ref[...]
    w0 = w[:, 0][:, None]
    w1c = w[:, 1][:, None]
    y0l, y0h = _unpack_cols(y0_ref[...])
    y1l, y1h = _unpack_cols(y1_ref[...])
    out_ref[:, :HD] = y0l * w0 + y1l * w1c
    out_ref[:, HD:] = y0h * w0 + y1h * w1c


_combine = pl.pallas_call(
    _combine_body,
    grid=(T // TBC,),
    in_specs=[
        pl.BlockSpec((TBC, HD), lambda i: (i, 0)),
        pl.BlockSpec((TBC, HD), lambda i: (i + T // TBC, 0)),
        pl.BlockSpec((TBC, K), lambda i: (i, 0)),
    ],
    out_specs=pl.BlockSpec((TBC, D), lambda i: (i, 0)),
    out_shape=jax.ShapeDtypeStruct((T, D), jnp.float32),
)


def kernel(x, expert_weights, expert_indices, w1, w2):
    counts, dest, block_expert, xi = _routing(expert_indices, x)
    tok_km = jnp.concatenate(
        [jnp.arange(T, dtype=jnp.int32), jnp.arange(T, dtype=jnp.int32)]
    )
    iota_r = jnp.arange(ROWS, dtype=jnp.int32)

    permute = _sc_kernels()
    xs = permute(xi, tok_km, dest)
    ys = _grouped_gemm(block_expert, xs, w1, w2)
    ykm = permute(ys, dest, iota_r)
    out = _combine(ykm, ykm, expert_weights.astype(jnp.float32))
    return out, counts


# speedup vs baseline: 1.0589x; 1.0589x over previous
import functools

import jax
import jax.numpy as jnp
from jax import lax
from jax.experimental import pallas as pl
from jax.experimental.pallas import tpu as pltpu
from jax.experimental.pallas import tpu_sc as plsc

E = 8
K = 2
T = 2048
D = 1024
F = 1024
HD = D // 2
ROWS = T * K
BLK = 256
NB = 23
NPAD = NB * BLK

RR = 32
RL = 128

NC = 2
NS = 16
NW = NC * NS
TPW = T // NW


def _bf16_hi_bits(f):
    b = lax.bitcast_convert_type(f, jnp.uint32)
    return b + jnp.uint32(0x7FFF) + ((b >> 16) & jnp.uint32(1))


def _pack_cols(lo_f, hi_f):
    lo = _bf16_hi_bits(lo_f) >> 16
    hi = _bf16_hi_bits(hi_f) & jnp.uint32(0xFFFF0000)
    return lax.bitcast_convert_type(lo | hi, jnp.int32)


def _unpack_cols(p_i32):
    p = lax.bitcast_convert_type(p_i32, jnp.uint32)
    lo = lax.bitcast_convert_type(p << 16, jnp.float32)
    hi = lax.bitcast_convert_type(p & jnp.uint32(0xFFFF0000), jnp.float32)
    return lo, hi


def _routing_body(fe_ref, counts_ref, dest_ref, be_ref):
    fe = fe_ref[...]
    dest = jnp.zeros((RR, RL), jnp.int32)
    counts = jnp.zeros((1, E), jnp.int32)
    bexp = jnp.zeros((1, NB), jnp.int32)
    lane_e = lax.broadcasted_iota(jnp.int32, (1, E), 1)
    lane_b = lax.broadcasted_iota(jnp.int32, (1, NB), 1)
    blk_start = jnp.int32(0)
    for e in range(E):
        m = (fe == e).astype(jnp.int32)
        ic = m
        for s in (1, 2, 4, 8, 16, 32, 64):
            ic = ic + jnp.concatenate(
                [jnp.zeros((RR, s), jnp.int32), ic[:, : RL - s]], axis=1
            )
        rt = ic[:, RL - 1 :]
        er = rt
        for s in (1, 2, 4, 8, 16):
            er = er + jnp.concatenate(
                [jnp.zeros((s, 1), jnp.int32), er[: RR - s, :]], axis=0
            )
        er = er - rt
        c_e = er[RR - 1, 0] + rt[RR - 1, 0]
        nblk_e = (c_e + BLK - 1) // BLK
        pad_base = blk_start * BLK
        rank_e = er + ic - 1
        dest = dest + m * (rank_e + pad_base)
        counts = counts + jnp.where(lane_e == e, c_e, 0)
        bexp = bexp + (lane_b >= blk_start).astype(jnp.int32)
        blk_start = blk_start + nblk_e
    counts_ref[...] = counts
    dest_ref[...] = dest
    be_ref[...] = jnp.clip(bexp - 1, 0, E - 1)


_routing_call = pl.pallas_call(
    _routing_body,
    out_shape=[
        jax.ShapeDtypeStruct((1, E), jnp.int32),
        jax.ShapeDtypeStruct((RR, RL), jnp.int32),
        jax.ShapeDtypeStruct((1, NB), jnp.int32),
    ],
)


def _routing(expert_indices):
    fe = expert_indices.reshape(RR, RL).astype(jnp.int32)
    counts, dest, block_expert = _routing_call(fe)
    return counts.reshape(E), dest.reshape(ROWS), block_expert.reshape(NB)


@functools.lru_cache(maxsize=None)
def _sc_kernels():
    mesh = plsc.VectorSubcoreMesh(core_axis_name="c", subcore_axis_name="s")

    @functools.partial(
        pl.kernel,
        mesh=mesh,
        out_type=jax.ShapeDtypeStruct((NPAD, D), jnp.float32),
        scratch_types=[
            pltpu.VMEM((TPW,), jnp.int32),
            pltpu.VMEM((TPW,), jnp.int32),
            pltpu.VMEM((TPW, D), jnp.float32),
            pltpu.SemaphoreType.DMA,
        ],
    )
    def permute_in(x_hbm, d0_hbm, d1_hbm, xs_hbm, d0_v, d1_v, xbuf, sem):
        wid = lax.axis_index("s") * NC + lax.axis_index("c")
        base = wid * TPW
        pltpu.sync_copy(d0_hbm.at[pl.ds(base, TPW)], d0_v)
        pltpu.sync_copy(d1_hbm.at[pl.ds(base, TPW)], d1_v)
        pltpu.sync_copy(x_hbm.at[pl.ds(base, TPW)], xbuf)
        c0 = pltpu.async_copy(xbuf, xs_hbm.at[d0_v], sem)
        c1 = pltpu.async_copy(xbuf, xs_hbm.at[d1_v], sem)
        c0.wait()
        c1.wait()

    @functools.partial(
        pl.kernel,
        mesh=mesh,
        out_type=[
            jax.ShapeDtypeStruct((T, HD), jnp.int32),
            jax.ShapeDtypeStruct((T, HD), jnp.int32),
        ],
        scratch_types=[
            pltpu.VMEM((TPW,), jnp.int32),
            pltpu.VMEM((TPW,), jnp.int32),
            pltpu.VMEM((TPW, HD), jnp.int32),
            pltpu.VMEM((TPW, HD), jnp.int32),
            pltpu.SemaphoreType.DMA,
        ],
    )
    def permute_out(ys_hbm, d0_hbm, d1_hbm, y0_hbm, y1_hbm,
                    d0_v, d1_v, buf0, buf1, sem):
        wid = lax.axis_index("s") * NC + lax.axis_index("c")
        base = wid * TPW
        pltpu.sync_copy(d0_hbm.at[pl.ds(base, TPW)], d0_v)
        pltpu.sync_copy(d1_hbm.at[pl.ds(base, TPW)], d1_v)
        ca = pltpu.async_copy(ys_hbm.at[d0_v], buf0, sem)
        cb = pltpu.async_copy(ys_hbm.at[d1_v], buf1, sem)
        ca.wait()
        cb.wait()
        pltpu.sync_copy(buf0, y0_hbm.at[pl.ds(base, TPW)])
        pltpu.sync_copy(buf1, y1_hbm.at[pl.ds(base, TPW)])

    return permute_in, permute_out


def _gemm_body(be_ref, xs_ref, w1_ref, w2_ref, ys_ref):
    h = jnp.maximum(
        jnp.dot(xs_ref[...], w1_ref[0], preferred_element_type=jnp.float32), 0.0
    )
    y = jnp.dot(h, w2_ref[0], preferred_element_type=jnp.float32)
    ys_ref[...] = _pack_cols(y[:, :HD], y[:, HD:])


_grouped_gemm = pl.pallas_call(
    _gemm_body,
    grid_spec=pltpu.PrefetchScalarGridSpec(
        num_scalar_prefetch=1,
        grid=(NB,),
        in_specs=[
            pl.BlockSpec((BLK, D), lambda b, be: (b, 0)),
            pl.BlockSpec((1, D, F), lambda b, be: (be[b], 0, 0)),
            pl.BlockSpec((1, F, D), lambda b, be: (be[b], 0, 0)),
        ],
        out_specs=pl.BlockSpec((BLK, HD), lambda b, be: (b, 0)),
    ),
    out_shape=jax.ShapeDtypeStruct((NPAD, HD), jnp.int32),
    compiler_params=pltpu.CompilerParams(
        dimension_semantics=("arbitrary",),
    ),
)


TBC = 256


def _combine_body(y0_ref, y1_ref, w_ref, out_ref):
    w = w_ref[...]
    w0 = w[:, 0][:, None]
    w1c = w[:, 1][:, None]
    y0l, y0h = _unpack_cols(y0_ref[...])
    y1l, y1h = _unpack_cols(y1_ref[...])
    out_ref[:, :HD] = y0l * w0 + y1l * w1c
    out_ref[:, HD:] = y0h * w0 + y1h * w1c


_combine = pl.pallas_call(
    _combine_body,
    grid=(T // TBC,),
    in_specs=[
        pl.BlockSpec((TBC, HD), lambda i: (i, 0)),
        pl.BlockSpec((TBC, HD), lambda i: (i, 0)),
        pl.BlockSpec((TBC, K), lambda i: (i, 0)),
    ],
    out_specs=pl.BlockSpec((TBC, D), lambda i: (i, 0)),
    out_shape=jax.ShapeDtypeStruct((T, D), jnp.float32),
)


def kernel(x, expert_weights, expert_indices, w1, w2):
    counts, dest, block_expert = _routing(expert_indices)
    dp = dest.reshape(T, K)
    d0 = dp[:, 0]
    d1 = dp[:, 1]

    permute_in, permute_out = _sc_kernels()
    xs = permute_in(x, d0, d1)
    ys = _grouped_gemm(block_expert, xs, w1, w2)
    y0i, y1i = permute_out(ys, d0, d1)
    out = _combine(y0i, y1i, expert_weights.astype(jnp.float32))
    return out, counts


# speedup vs baseline: 1.0700x; 1.0105x over previous
import functools

import jax
import jax.numpy as jnp
from jax import lax
from jax.experimental import pallas as pl
from jax.experimental.pallas import tpu as pltpu
from jax.experimental.pallas import tpu_sc as plsc

E = 8
K = 2
T = 2048
D = 1024
F = 1024
HD = D // 2
ROWS = T * K
BLK = 256
NB = 23
NPAD = NB * BLK

RR = 32
RL = 128

NC = 2
NS = 16
NW = NC * NS
TPW = T // NW


def _bf16_hi_bits(f):
    b = lax.bitcast_convert_type(f, jnp.uint32)
    return b + jnp.uint32(0x7FFF) + ((b >> 16) & jnp.uint32(1))


def _pack_cols(lo_f, hi_f):
    lo = _bf16_hi_bits(lo_f) >> 16
    hi = _bf16_hi_bits(hi_f) & jnp.uint32(0xFFFF0000)
    return lax.bitcast_convert_type(lo | hi, jnp.int32)


def _unpack_cols(p_i32):
    p = lax.bitcast_convert_type(p_i32, jnp.uint32)
    lo = lax.bitcast_convert_type(p << 16, jnp.float32)
    hi = lax.bitcast_convert_type(p & jnp.uint32(0xFFFF0000), jnp.float32)
    return lo, hi


def _routing_body(fe_ref, counts_ref, dest_ref, be_ref):
    fe = fe_ref[...]
    dest = jnp.zeros((RR, RL), jnp.int32)
    counts = jnp.zeros((1, E), jnp.int32)
    bexp = jnp.zeros((1, NB), jnp.int32)
    lane_e = lax.broadcasted_iota(jnp.int32, (1, E), 1)
    lane_b = lax.broadcasted_iota(jnp.int32, (1, NB), 1)
    blk_start = jnp.int32(0)
    for e in range(E):
        m = (fe == e).astype(jnp.int32)
        ic = m
        for s in (1, 2, 4, 8, 16, 32, 64):
            ic = ic + jnp.concatenate(
                [jnp.zeros((RR, s), jnp.int32), ic[:, : RL - s]], axis=1
            )
        rt = ic[:, RL - 1 :]
        er = rt
        for s in (1, 2, 4, 8, 16):
            er = er + jnp.concatenate(
                [jnp.zeros((s, 1), jnp.int32), er[: RR - s, :]], axis=0
            )
        er = er - rt
        c_e = er[RR - 1, 0] + rt[RR - 1, 0]
        nblk_e = (c_e + BLK - 1) // BLK
        pad_base = blk_start * BLK
        rank_e = er + ic - 1
        dest = dest + m * (rank_e + pad_base)
        counts = counts + jnp.where(lane_e == e, c_e, 0)
        bexp = bexp + (lane_b >= blk_start).astype(jnp.int32)
        blk_start = blk_start + nblk_e
    counts_ref[...] = counts
    dest_ref[...] = dest
    be_ref[...] = jnp.clip(bexp - 1, 0, E - 1)


_routing_call = pl.pallas_call(
    _routing_body,
    out_shape=[
        jax.ShapeDtypeStruct((1, E), jnp.int32),
        jax.ShapeDtypeStruct((RR, RL), jnp.int32),
        jax.ShapeDtypeStruct((1, NB), jnp.int32),
    ],
)


def _routing(expert_indices):
    fe = expert_indices.astype(jnp.int32).T.reshape(RR, RL)
    counts, dest, block_expert = _routing_call(fe)
    return counts.reshape(E), dest.reshape(ROWS), block_expert.reshape(NB)


@functools.lru_cache(maxsize=None)
def _sc_kernels():
    mesh = plsc.VectorSubcoreMesh(core_axis_name="c", subcore_axis_name="s")

    @functools.partial(
        pl.kernel,
        mesh=mesh,
        out_type=jax.ShapeDtypeStruct((NPAD, D), jnp.float32),
        scratch_types=[
            pltpu.VMEM((TPW,), jnp.int32),
            pltpu.VMEM((TPW,), jnp.int32),
            pltpu.VMEM((TPW, D), jnp.float32),
            pltpu.SemaphoreType.DMA,
        ],
    )
    def permute_in(x_hbm, d0_hbm, d1_hbm, xs_hbm, d0_v, d1_v, xbuf, sem):
        wid = lax.axis_index("s") * NC + lax.axis_index("c")
        base = wid * TPW
        pltpu.sync_copy(d0_hbm.at[pl.ds(base, TPW)], d0_v)
        pltpu.sync_copy(d1_hbm.at[pl.ds(base, TPW)], d1_v)
        pltpu.sync_copy(x_hbm.at[pl.ds(base, TPW)], xbuf)
        c0 = pltpu.async_copy(xbuf, xs_hbm.at[d0_v], sem)
        c1 = pltpu.async_copy(xbuf, xs_hbm.at[d1_v], sem)
        c0.wait()
        c1.wait()

    @functools.partial(
        pl.kernel,
        mesh=mesh,
        out_type=[
            jax.ShapeDtypeStruct((T, HD), jnp.int32),
            jax.ShapeDtypeStruct((T, HD), jnp.int32),
        ],
        scratch_types=[
            pltpu.VMEM((TPW,), jnp.int32),
            pltpu.VMEM((TPW,), jnp.int32),
            pltpu.VMEM((TPW, HD), jnp.int32),
            pltpu.VMEM((TPW, HD), jnp.int32),
            pltpu.SemaphoreType.DMA,
        ],
    )
    def permute_out(ys_hbm, d0_hbm, d1_hbm, y0_hbm, y1_hbm,
                    d0_v, d1_v, buf0, buf1, sem):
        wid = lax.axis_index("s") * NC + lax.axis_index("c")
        base = wid * TPW
        pltpu.sync_copy(d0_hbm.at[pl.ds(base, TPW)], d0_v)
        pltpu.sync_copy(d1_hbm.at[pl.ds(base, TPW)], d1_v)
        ca = pltpu.async_copy(ys_hbm.at[d0_v], buf0, sem)
        cb = pltpu.async_copy(ys_hbm.at[d1_v], buf1, sem)
        ca.wait()
        cb.wait()
        pltpu.sync_copy(buf0, y0_hbm.at[pl.ds(base, TPW)])
        pltpu.sync_copy(buf1, y1_hbm.at[pl.ds(base, TPW)])

    return permute_in, permute_out


def _gemm_body(be_ref, xs_ref, w1_ref, w2_ref, ys_ref):
    h = jnp.maximum(
        jnp.dot(xs_ref[...], w1_ref[0], preferred_element_type=jnp.float32), 0.0
    )
    y = jnp.dot(h, w2_ref[0], preferred_element_type=jnp.float32)
    ys_ref[...] = _pack_cols(y[:, :HD], y[:, HD:])


_grouped_gemm = pl.pallas_call(
    _gemm_body,
    grid_spec=pltpu.PrefetchScalarGridSpec(
        num_scalar_prefetch=1,
        grid=(NB,),
        in_specs=[
            pl.BlockSpec((BLK, D), lambda b, be: (b, 0)),
            pl.BlockSpec((1, D, F), lambda b, be: (be[b], 0, 0)),
            pl.BlockSpec((1, F, D), lambda b, be: (be[b], 0, 0)),
        ],
        out_specs=pl.BlockSpec((BLK, HD), lambda b, be: (b, 0)),
    ),
    out_shape=jax.ShapeDtypeStruct((NPAD, HD), jnp.int32),
    compiler_params=pltpu.CompilerParams(
        dimension_semantics=("arbitrary",),
    ),
)


TBC = 256


def _combine_body(y0_ref, y1_ref, w_ref, out_ref):
    w = w_ref[...]
    w0 = w[:, 0][:, None]
    w1c = w[:, 1][:, None]
    y0l, y0h = _unpack_cols(y0_ref[...])
    y1l, y1h = _unpack_cols(y1_ref[...])
    out_ref[:, :HD] = y0l * w0 + y1l * w1c
    out_ref[:, HD:] = y0h * w0 + y1h * w1c


_combine = pl.pallas_call(
    _combine_body,
    grid=(T // TBC,),
    in_specs=[
        pl.BlockSpec((TBC, HD), lambda i: (i, 0)),
        pl.BlockSpec((TBC, HD), lambda i: (i, 0)),
        pl.BlockSpec((TBC, K), lambda i: (i, 0)),
    ],
    out_specs=pl.BlockSpec((TBC, D), lambda i: (i, 0)),
    out_shape=jax.ShapeDtypeStruct((T, D), jnp.float32),
)


def kernel(x, expert_weights, expert_indices, w1, w2):
    counts, dest, block_expert = _routing(expert_indices)
    d0 = dest[:T]
    d1 = dest[T:]

    permute_in, permute_out = _sc_kernels()
    xs = permute_in(x, d0, d1)
    ys = _grouped_gemm(block_expert, xs, w1, w2)
    y0i, y1i = permute_out(ys, d0, d1)
    out = _combine(y0i, y1i, expert_weights.astype(jnp.float32))
    return out, counts


# speedup vs baseline: 1.1032x; 1.0310x over previous
import functools

import jax
import jax.numpy as jnp
from jax import lax
from jax.experimental import pallas as pl
from jax.experimental.pallas import tpu as pltpu
from jax.experimental.pallas import tpu_sc as plsc

E = 8
K = 2
T = 2048
D = 1024
F = 1024
HD = D // 2
ROWS = T * K
BLK = 256
NB = 23
NPAD = NB * BLK

RR = 32
RL = 128

NC = 2
NS = 16
NW = NC * NS
TPW = T // NW


def _bf16_hi_bits(f):
    b = lax.bitcast_convert_type(f, jnp.uint32)
    return b + jnp.uint32(0x7FFF) + ((b >> 16) & jnp.uint32(1))


def _pack_cols(lo_f, hi_f):
    lo = _bf16_hi_bits(lo_f) >> 16
    hi = _bf16_hi_bits(hi_f) & jnp.uint32(0xFFFF0000)
    return lax.bitcast_convert_type(lo | hi, jnp.int32)


def _unpack_cols(p_i32):
    p = lax.bitcast_convert_type(p_i32, jnp.uint32)
    lo = lax.bitcast_convert_type(p << 16, jnp.float32)
    hi = lax.bitcast_convert_type(p & jnp.uint32(0xFFFF0000), jnp.float32)
    return lo, hi


def _routing_body(fe_ref, counts_ref, dest_ref, be_ref):
    fe = fe_ref[...]
    dest = jnp.zeros((RR, RL), jnp.int32)
    counts = jnp.zeros((1, E), jnp.int32)
    bexp = jnp.zeros((1, NB), jnp.int32)
    lane_e = lax.broadcasted_iota(jnp.int32, (1, E), 1)
    lane_b = lax.broadcasted_iota(jnp.int32, (1, NB), 1)
    blk_start = jnp.int32(0)
    for e in range(E):
        m = (fe == e).astype(jnp.int32)
        ic = m
        for s in (1, 2, 4, 8, 16, 32, 64):
            ic = ic + jnp.concatenate(
                [jnp.zeros((RR, s), jnp.int32), ic[:, : RL - s]], axis=1
            )
        rt = ic[:, RL - 1 :]
        er = rt
        for s in (1, 2, 4, 8, 16):
            er = er + jnp.concatenate(
                [jnp.zeros((s, 1), jnp.int32), er[: RR - s, :]], axis=0
            )
        er = er - rt
        c_e = er[RR - 1, 0] + rt[RR - 1, 0]
        nblk_e = (c_e + BLK - 1) // BLK
        pad_base = blk_start * BLK
        rank_e = er + ic - 1
        dest = dest + m * (rank_e + pad_base)
        counts = counts + jnp.where(lane_e == e, c_e, 0)
        bexp = bexp + (lane_b >= blk_start).astype(jnp.int32)
        blk_start = blk_start + nblk_e
    counts_ref[...] = counts
    dest_ref[...] = dest
    be_ref[...] = jnp.clip(bexp - 1, 0, E - 1)


_routing_call = pl.pallas_call(
    _routing_body,
    out_shape=[
        jax.ShapeDtypeStruct((1, E), jnp.int32),
        jax.ShapeDtypeStruct((RR, RL), jnp.int32),
        jax.ShapeDtypeStruct((1, NB), jnp.int32),
    ],
)


def _routing(expert_indices):
    fe = expert_indices.astype(jnp.int32).T.reshape(RR, RL)
    counts, dest, block_expert = _routing_call(fe)
    return counts.reshape(E), dest.reshape(ROWS), block_expert.reshape(NB)


@functools.lru_cache(maxsize=None)
def _sc_kernels():
    mesh = plsc.VectorSubcoreMesh(core_axis_name="c", subcore_axis_name="s")

    @functools.partial(
        pl.kernel,
        mesh=mesh,
        out_type=jax.ShapeDtypeStruct((NPAD, D), jnp.float32),
        scratch_types=[
            pltpu.VMEM((TPW,), jnp.int32),
            pltpu.VMEM((TPW,), jnp.int32),
            pltpu.VMEM((TPW, D), jnp.float32),
            pltpu.SemaphoreType.DMA,
        ],
    )
    def permute_in(x_hbm, dest_hbm, xs_hbm, d0_v, d1_v, xbuf, sem):
        wid = lax.axis_index("s") * NC + lax.axis_index("c")
        base = wid * TPW
        pltpu.sync_copy(dest_hbm.at[pl.ds(base, TPW)], d0_v)
        pltpu.sync_copy(dest_hbm.at[pl.ds(T + base, TPW)], d1_v)
        pltpu.sync_copy(x_hbm.at[pl.ds(base, TPW)], xbuf)
        c0 = pltpu.async_copy(xbuf, xs_hbm.at[d0_v], sem)
        c1 = pltpu.async_copy(xbuf, xs_hbm.at[d1_v], sem)
        c0.wait()
        c1.wait()

    @functools.partial(
        pl.kernel,
        mesh=mesh,
        out_type=[
            jax.ShapeDtypeStruct((T, HD), jnp.int32),
            jax.ShapeDtypeStruct((T, HD), jnp.int32),
        ],
        scratch_types=[
            pltpu.VMEM((TPW,), jnp.int32),
            pltpu.VMEM((TPW,), jnp.int32),
            pltpu.VMEM((TPW, HD), jnp.int32),
            pltpu.VMEM((TPW, HD), jnp.int32),
            pltpu.SemaphoreType.DMA,
        ],
    )
    def permute_out(ys_hbm, dest_hbm, y0_hbm, y1_hbm,
                    d0_v, d1_v, buf0, buf1, sem):
        wid = lax.axis_index("s") * NC + lax.axis_index("c")
        base = wid * TPW
        pltpu.sync_copy(dest_hbm.at[pl.ds(base, TPW)], d0_v)
        pltpu.sync_copy(dest_hbm.at[pl.ds(T + base, TPW)], d1_v)
        ca = pltpu.async_copy(ys_hbm.at[d0_v], buf0, sem)
        cb = pltpu.async_copy(ys_hbm.at[d1_v], buf1, sem)
        ca.wait()
        cb.wait()
        pltpu.sync_copy(buf0, y0_hbm.at[pl.ds(base, TPW)])
        pltpu.sync_copy(buf1, y1_hbm.at[pl.ds(base, TPW)])

    return permute_in, permute_out


def _gemm_body(be_ref, xs_ref, w1_ref, w2_ref, ys_ref):
    h = jnp.maximum(
        jnp.dot(xs_ref[...], w1_ref[0], preferred_element_type=jnp.float32), 0.0
    )
    y = jnp.dot(h, w2_ref[0], preferred_element_type=jnp.float32)
    ys_ref[...] = _pack_cols(y[:, :HD], y[:, HD:])


_grouped_gemm = pl.pallas_call(
    _gemm_body,
    grid_spec=pltpu.PrefetchScalarGridSpec(
        num_scalar_prefetch=1,
        grid=(NB,),
        in_specs=[
            pl.BlockSpec((BLK, D), lambda b, be: (b, 0)),
            pl.BlockSpec((1, D, F), lambda b, be: (be[b], 0, 0)),
            pl.BlockSpec((1, F, D), lambda b, be: (be[b], 0, 0)),
        ],
        out_specs=pl.BlockSpec((BLK, HD), lambda b, be: (b, 0)),
    ),
    out_shape=jax.ShapeDtypeStruct((NPAD, HD), jnp.int32),
    compiler_params=pltpu.CompilerParams(
        dimension_semantics=("arbitrary",),
    ),
)


TBC = 256


def _combine_body(y0_ref, y1_ref, w_ref, out_ref):
    w = w_ref[...]
    w0 = w[:, 0][:, None]
    w1c = w[:, 1][:, None]
    y0l, y0h = _unpack_cols(y0_ref[...])
    y1l, y1h = _unpack_cols(y1_ref[...])
    out_ref[:, :HD] = y0l * w0 + y1l * w1c
    out_ref[:, HD:] = y0h * w0 + y1h * w1c


_combine = pl.pallas_call(
    _combine_body,
    grid=(T // TBC,),
    in_specs=[
        pl.BlockSpec((TBC, HD), lambda i: (i, 0)),
        pl.BlockSpec((TBC, HD), lambda i: (i, 0)),
        pl.BlockSpec((TBC, K), lambda i: (i, 0)),
    ],
    out_specs=pl.BlockSpec((TBC, D), lambda i: (i, 0)),
    out_shape=jax.ShapeDtypeStruct((T, D), jnp.float32),
)


def kernel(x, expert_weights, expert_indices, w1, w2):
    counts, dest, block_expert = _routing(expert_indices)

    permute_in, permute_out = _sc_kernels()
    xs = permute_in(x, dest)
    ys = _grouped_gemm(block_expert, xs, w1, w2)
    y0i, y1i = permute_out(ys, dest)
    out = _combine(y0i, y1i, expert_weights.astype(jnp.float32))
    return out, counts
